# baseline (speedup 1.0000x reference)
import jax
import jax.numpy as jnp
from jax.experimental import pallas as pl
from jax.experimental.pallas import tpu as pltpu


def _round_up(n, m):
    return ((n + m - 1) // m) * m


def _lstm_kernel(x_ref,
                 wih_ref,
                 whh_ref,
                 w1_ref,
                 b1_ref,
                 w2_ref,
                 b2_ref,
                 out_ref):
    T, BB, D = x_ref.shape
    H = whh_ref.shape[0]

    wih = wih_ref[...]
    whh = whh_ref[...]

    def step(gates, c, first):
        th_ig = jnp.tanh(gates[:, :2 * H])
        m = th_ig[:, :H] * th_ig[:, H:] + th_ig[:, H:]
        if first:
            c = 0.5 * m
        else:
            th_f = jnp.tanh(gates[:, 2 * H:3 * H])
            c = 0.5 * (th_f * c + c + m)
        th_o = jnp.tanh(gates[:, 3 * H:])
        h2 = jnp.tanh(c) * (th_o + 1.0)
        return h2, c

    g0 = jnp.dot(x_ref[0], wih, preferred_element_type=jnp.float32)
    h2, c = step(g0, None, True)
    h2_bf = h2.astype(jnp.bfloat16)

    for t in range(1, T):
        gates = (jnp.dot(x_ref[t], wih, preferred_element_type=jnp.float32)
                 + jnp.dot(h2_bf, whh, preferred_element_type=jnp.float32))
        h2, c = step(gates, c, False)
        h2_bf = h2.astype(jnp.bfloat16)

    z1 = jnp.dot(h2, w1_ref[...], preferred_element_type=jnp.float32) + b1_ref[...]
    z1 = jnp.maximum(z1, 0.0)
    z2 = jnp.dot(z1, w2_ref[...], preferred_element_type=jnp.float32) + b2_ref[...]
    out_ref[...] = jax.nn.sigmoid(z2)


def kernel(x, wih_t, whh_t, b_lstm, w1_t, b1, w2_t, b2):
    B, T, D = x.shape
    H = whh_t.shape[0]
    G = 4 * H
    F1 = w1_t.shape[1]
    O = w2_t.shape[1]

    batch_block = min(512, _round_up(B, 8))
    batch_block = max(8, _round_up(batch_block, 8))
    B_pad = _round_up(B, batch_block)
    OP = _round_up(O, 128)
    if B_pad != B:
        x = jnp.pad(x, ((0, B_pad - B), (0, 0), (0, 0)))
    w2p = jnp.pad(w2_t, ((0, 0), (0, OP - O)))
    b2p = jnp.pad(b2, ((0, 0), (0, OP - O)))

    lane = jnp.arange(G)
    g_lane = (lane >= 2 * H) & (lane < 3 * H)
    colscale = jnp.where(g_lane, 1.0, 0.5).astype(jnp.float32)

    def reorder(w):
        return jnp.concatenate(
            [w[:, :H], w[:, 2 * H:3 * H], w[:, H:2 * H], w[:, 3 * H:]], axis=1)

    wih_aug = jnp.concatenate(
        [wih_t, b_lstm, jnp.zeros((7, G), jnp.float32)], axis=0)
    wih_s = reorder(wih_aug * colscale[None, :]).astype(jnp.bfloat16)
    whh_s = reorder(0.5 * whh_t * colscale[None, :]).astype(jnp.bfloat16)
    w1_s = 0.5 * w1_t

    x_tm = jnp.transpose(x, (1, 0, 2)).astype(jnp.bfloat16)
    hot = jnp.concatenate(
        [jnp.ones((T, B_pad, 1), jnp.bfloat16),
         jnp.zeros((T, B_pad, 7), jnp.bfloat16)], axis=2)
    x_tm = jnp.concatenate([x_tm, hot], axis=2)
    nb = B_pad // batch_block

    out = pl.pallas_call(
        _lstm_kernel,
        out_shape=jax.ShapeDtypeStruct((B_pad, OP), jnp.float32),
        grid_spec=pltpu.PrefetchScalarGridSpec(
            num_scalar_prefetch=0,
            grid=(nb,),
            in_specs=[
                pl.BlockSpec((T, batch_block, D + 8), lambda i: (0, i, 0)),
                pl.BlockSpec((D + 8, G), lambda i: (0, 0)),
                pl.BlockSpec((H, G), lambda i: (0, 0)),
                pl.BlockSpec((H, F1), lambda i: (0, 0)),
                pl.BlockSpec((1, F1), lambda i: (0, 0)),
                pl.BlockSpec((F1, OP), lambda i: (0, 0)),
                pl.BlockSpec((1, OP), lambda i: (0, 0)),
            ],
            out_specs=pl.BlockSpec((batch_block, OP), lambda i: (i, 0)),
        ),
        compiler_params=pltpu.CompilerParams(
            dimension_semantics=("parallel",),
            vmem_limit_bytes=100 * 1024 * 1024,
        ),
    )(x_tm, wih_s, whh_s, w1_s, b1, w2p, b2p)

    return out[:B, :O]


# speedup vs baseline: 1.1470x; 1.1470x over previous
import jax
import jax.numpy as jnp
from jax.experimental import pallas as pl
from jax.experimental.pallas import tpu as pltpu


def _round_up(n, m):
    return ((n + m - 1) // m) * m


def _lstm_kernel(x_ref,
                 wih_ref,
                 whh_ref,
                 b_ref,
                 w1_ref,
                 b1_ref,
                 w2_ref,
                 b2_ref,
                 out_ref,
                 h2_s,
                 c_s):
    BB, TC, D = x_ref.shape
    H = whh_ref.shape[0]
    j = pl.program_id(1)
    NT = pl.num_programs(1)

    wih = wih_ref[...]
    whh = whh_ref[...]
    bias = b_ref[...]

    xt = jnp.transpose(x_ref[...], (1, 0, 2)).astype(jnp.bfloat16)

    @pl.when(j == 0)
    def _init():
        h2_s[...] = jnp.zeros_like(h2_s)
        c_s[...] = jnp.zeros_like(c_s)

    def step(gates, c):
        th_ig = jnp.tanh(gates[:, :2 * H])
        m = th_ig[:, :H] * th_ig[:, H:] + th_ig[:, H:]
        th_f = jnp.tanh(gates[:, 2 * H:3 * H])
        c = 0.5 * (th_f * c + c + m)
        th_o = jnp.tanh(gates[:, 3 * H:])
        h2 = jnp.tanh(c) * (th_o + 1.0)
        return h2, c

    h2_bf = h2_s[...]
    c = c_s[...]
    h2 = None
    for k in range(TC):
        gates = (jnp.dot(xt[k], wih, preferred_element_type=jnp.float32)
                 + jnp.dot(h2_bf, whh, preferred_element_type=jnp.float32)
                 + bias)
        h2, c = step(gates, c)
        h2_bf = h2.astype(jnp.bfloat16)
    h2_s[...] = h2_bf
    c_s[...] = c

    @pl.when(j == NT - 1)
    def _head():
        z1 = (jnp.dot(h2, w1_ref[...], preferred_element_type=jnp.float32)
              + b1_ref[...])
        z1 = jnp.maximum(z1, 0.0)
        z2 = (jnp.dot(z1, w2_ref[...], preferred_element_type=jnp.float32)
              + b2_ref[...])
        out_ref[...] = jax.nn.sigmoid(z2)


def kernel(x, wih_t, whh_t, b_lstm, w1_t, b1, w2_t, b2):
    B, T, D = x.shape
    H = whh_t.shape[0]
    G = 4 * H
    F1 = w1_t.shape[1]
    O = w2_t.shape[1]

    batch_block = min(512, _round_up(B, 8))
    batch_block = max(8, _round_up(batch_block, 8))
    B_pad = _round_up(B, batch_block)
    OP = _round_up(O, 128)
    if B_pad != B:
        x = jnp.pad(x, ((0, B_pad - B), (0, 0), (0, 0)))
    w2p = jnp.pad(w2_t, ((0, 0), (0, OP - O)))
    b2p = jnp.pad(b2, ((0, 0), (0, OP - O)))

    TC = 8
    while T % TC:
        TC -= 1
    NT = T // TC

    lane = jnp.arange(G)
    g_lane = (lane >= 2 * H) & (lane < 3 * H)
    colscale = jnp.where(g_lane, 1.0, 0.5).astype(jnp.float32)

    def reorder(w):
        return jnp.concatenate(
            [w[:, :H], w[:, 2 * H:3 * H], w[:, H:2 * H], w[:, 3 * H:]], axis=1)

    wih_s = reorder(wih_t * colscale[None, :]).astype(jnp.bfloat16)
    whh_s = reorder(0.5 * whh_t * colscale[None, :]).astype(jnp.bfloat16)
    b_s = reorder(b_lstm * colscale[None, :])
    w1_s = 0.5 * w1_t

    nb = B_pad // batch_block

    out = pl.pallas_call(
        _lstm_kernel,
        out_shape=jax.ShapeDtypeStruct((B_pad, OP), jnp.float32),
        grid_spec=pltpu.PrefetchScalarGridSpec(
            num_scalar_prefetch=0,
            grid=(nb, NT),
            in_specs=[
                pl.BlockSpec((batch_block, TC, D), lambda i, j: (i, j, 0)),
                pl.BlockSpec((D, G), lambda i, j: (0, 0)),
                pl.BlockSpec((H, G), lambda i, j: (0, 0)),
                pl.BlockSpec((1, G), lambda i, j: (0, 0)),
                pl.BlockSpec((H, F1), lambda i, j: (0, 0)),
                pl.BlockSpec((1, F1), lambda i, j: (0, 0)),
                pl.BlockSpec((F1, OP), lambda i, j: (0, 0)),
                pl.BlockSpec((1, OP), lambda i, j: (0, 0)),
            ],
            out_specs=pl.BlockSpec((batch_block, OP), lambda i, j: (i, 0)),
            scratch_shapes=[pltpu.VMEM((batch_block, H), jnp.bfloat16),
                            pltpu.VMEM((batch_block, H), jnp.float32)],
        ),
        compiler_params=pltpu.CompilerParams(
            dimension_semantics=("parallel", "arbitrary"),
            vmem_limit_bytes=100 * 1024 * 1024,
        ),
    )(x, wih_s, whh_s, b_s, w1_s, b1, w2p, b2p)

    return out[:B, :O]


# speedup vs baseline: 1.1621x; 1.0131x over previous
import jax
import jax.numpy as jnp
from jax.experimental import pallas as pl
from jax.experimental.pallas import tpu as pltpu


def _round_up(n, m):
    return ((n + m - 1) // m) * m


def _lstm_kernel(x_ref,
                 wih_ref,
                 whh_ref,
                 b_ref,
                 w1_ref,
                 b1_ref,
                 w2_ref,
                 b2_ref,
                 out_ref,
                 h2_s,
                 c_s):
    BB, TC, D = x_ref.shape
    H = whh_ref.shape[0]
    j = pl.program_id(1)
    NT = pl.num_programs(1)

    wih = wih_ref[...]
    whh = whh_ref[...]
    bias = b_ref[...]

    xt = jnp.transpose(x_ref[...], (1, 0, 2)).astype(jnp.bfloat16)

    @pl.when(j == 0)
    def _init():
        h2_s[...] = jnp.zeros_like(h2_s)
        c_s[...] = jnp.zeros_like(c_s)

    def step(gates, c):
        th_ig = jnp.tanh(gates[:, :2 * H])
        m = th_ig[:, :H] * th_ig[:, H:] + th_ig[:, H:]
        th_f = jnp.tanh(gates[:, 2 * H:3 * H])
        c = 0.5 * (th_f * c + c + m)
        th_o = jnp.tanh(gates[:, 3 * H:])
        h2 = jnp.tanh(c) * (th_o + 1.0)
        return h2, c

    h2_bf = h2_s[...]
    c = c_s[...]
    h2 = None
    for k in range(TC):
        gates = (jnp.dot(xt[k], wih, preferred_element_type=jnp.float32)
                 + jnp.dot(h2_bf, whh, preferred_element_type=jnp.float32)
                 + bias)
        h2, c = step(gates, c)
        h2_bf = h2.astype(jnp.bfloat16)
    h2_s[...] = h2_bf
    c_s[...] = c

    @pl.when(j == NT - 1)
    def _head():
        z1 = (jnp.dot(h2, w1_ref[...], preferred_element_type=jnp.float32)
              + b1_ref[...])
        z1 = jnp.maximum(z1, 0.0)
        z2 = (jnp.dot(z1, w2_ref[...], preferred_element_type=jnp.float32)
              + b2_ref[...])
        out_ref[...] = jax.nn.sigmoid(z2)


def kernel(x, wih_t, whh_t, b_lstm, w1_t, b1, w2_t, b2):
    B, T, D = x.shape
    H = whh_t.shape[0]
    G = 4 * H
    F1 = w1_t.shape[1]
    O = w2_t.shape[1]

    batch_block = min(512, _round_up(B, 8))
    batch_block = max(8, _round_up(batch_block, 8))
    B_pad = _round_up(B, batch_block)
    OP = _round_up(O, 128)
    if B_pad != B:
        x = jnp.pad(x, ((0, B_pad - B), (0, 0), (0, 0)))
    w2p = jnp.pad(w2_t, ((0, 0), (0, OP - O)))
    b2p = jnp.pad(b2, ((0, 0), (0, OP - O)))

    TC = 16
    while T % TC:
        TC -= 1
    NT = T // TC

    lane = jnp.arange(G)
    g_lane = (lane >= 2 * H) & (lane < 3 * H)
    colscale = jnp.where(g_lane, 1.0, 0.5).astype(jnp.float32)

    def reorder(w):
        return jnp.concatenate(
            [w[:, :H], w[:, 2 * H:3 * H], w[:, H:2 * H], w[:, 3 * H:]], axis=1)

    wih_s = reorder(wih_t * colscale[None, :]).astype(jnp.bfloat16)
    whh_s = reorder(0.5 * whh_t * colscale[None, :]).astype(jnp.bfloat16)
    b_s = reorder(b_lstm * colscale[None, :])
    w1_s = 0.5 * w1_t

    nb = B_pad // batch_block

    out = pl.pallas_call(
        _lstm_kernel,
        out_shape=jax.ShapeDtypeStruct((B_pad, OP), jnp.float32),
        grid_spec=pltpu.PrefetchScalarGridSpec(
            num_scalar_prefetch=0,
            grid=(nb, NT),
            in_specs=[
                pl.BlockSpec((batch_block, TC, D), lambda i, j: (i, j, 0)),
                pl.BlockSpec((D, G), lambda i, j: (0, 0)),
                pl.BlockSpec((H, G), lambda i, j: (0, 0)),
                pl.BlockSpec((1, G), lambda i, j: (0, 0)),
                pl.BlockSpec((H, F1), lambda i, j: (0, 0)),
                pl.BlockSpec((1, F1), lambda i, j: (0, 0)),
                pl.BlockSpec((F1, OP), lambda i, j: (0, 0)),
                pl.BlockSpec((1, OP), lambda i, j: (0, 0)),
            ],
            out_specs=pl.BlockSpec((batch_block, OP), lambda i, j: (i, 0)),
            scratch_shapes=[pltpu.VMEM((batch_block, H), jnp.bfloat16),
                            pltpu.VMEM((batch_block, H), jnp.float32)],
        ),
        compiler_params=pltpu.CompilerParams(
            dimension_semantics=("parallel", "arbitrary"),
            vmem_limit_bytes=100 * 1024 * 1024,
        ),
    )(x, wih_s, whh_s, b_s, w1_s, b1, w2p, b2p)

    return out[:B, :O]
